# baseline (speedup 1.0000x reference)
import functools

import jax
import jax.numpy as jnp
from jax import lax
from jax.experimental import pallas as pl
from jax.experimental.pallas import tpu as pltpu
from jax.experimental.pallas import tpu_sc as plsc

N = 4096
E = 131072
D = 256
L = 3
G = 64
EPS = 1e-9
LOG_EPS = 1e-4

NC = 2
NS = 16
NW = NC * NS

RA = 24
PASS_STRIDE = RA * NW
NPASS = 6
ECHUNK = 4096
NECHUNK = E // ECHUNK

_f32 = jnp.float32


def _sc_build_a_body(edge_hbm, a_hbm, hist, ebuf0, ebuf1, esem0, esem1):
    c = lax.axis_index("c")
    s = lax.axis_index("s")
    gid = s * NC + c
    ones = jnp.full((16,), 1.0, _f32)
    z16 = jnp.zeros((16,), _f32)
    ebufs = (ebuf0, ebuf1)
    esems = (esem0, esem1)

    def chunk_off(ch):
        return pl.multiple_of(((ch + gid) % NECHUNK) * ECHUNK, 8)

    def start_chunk(ch, b):
        pltpu.make_async_copy(
            edge_hbm.at[:, pl.ds(chunk_off(ch), ECHUNK)], ebufs[b],
            esems[b]).start()

    def wait_chunk(ch, b):
        pltpu.make_async_copy(
            edge_hbm.at[:, pl.ds(chunk_off(ch), ECHUNK)], ebufs[b],
            esems[b]).wait()

    def one_pass(p, _):
        lo = jnp.minimum(p * PASS_STRIDE + gid * RA, N - RA)
        for r in range(RA):
            def zb(j, _2, r=r):
                for u in range(8):
                    hist[r, pl.ds(j * 128 + u * 16, 16)] = z16
                return 0
            lax.fori_loop(0, N // 128, zb, 0)
        start_chunk(0, 0)
        for ch in range(NECHUNK):
            b = ch % 2
            if ch + 1 < NECHUNK:
                start_chunk(ch + 1, (ch + 1) % 2)
            wait_chunk(ch, b)
            ebuf = ebufs[b]

            def scan(k, _2, ebuf=ebuf):
                for u in range(4):
                    s16 = ebuf[0, pl.ds(k * 64 + u * 16, 16)]
                    d16 = ebuf[1, pl.ds(k * 64 + u * 16, 16)]
                    rel = d16 - lo
                    mask = rel.astype(jnp.uint32) < jnp.uint32(RA)
                    plsc.addupdate_scatter(hist, [rel, s16], ones, mask=mask)
                return 0
            lax.fori_loop(0, ECHUNK // 64, scan, 0)
        pltpu.sync_copy(hist, a_hbm.at[pl.ds(lo, RA)])
        return 0

    lax.fori_loop(0, NPASS, one_pass, 0)


@functools.lru_cache(maxsize=1)
def _sc_build_a_kernel():
    return pl.kernel(
        _sc_build_a_body,
        out_type=jax.ShapeDtypeStruct((N, N), _f32),
        mesh=plsc.VectorSubcoreMesh(core_axis_name="c", subcore_axis_name="s",
                                    num_cores=NC, num_subcores=NS),
        compiler_params=pltpu.CompilerParams(needs_layout_passes=False),
        scratch_types=[
            pltpu.VMEM((RA, N), _f32),
            pltpu.VMEM((2, ECHUNK), jnp.int32),
            pltpu.VMEM((2, ECHUNK), jnp.int32),
            pltpu.SemaphoreType.DMA,
            pltpu.SemaphoreType.DMA,
        ],
    )


def _sc_build_a(edge_index):
    return _sc_build_a_kernel()(edge_index)


RBK = 512
NBLK = N // RBK


def _tc_gnn_body(a_ref, x_ref, w1_ref, b1_ref, w2_ref, b2_ref, o_ref, s0, s1):
    l = pl.program_id(0)
    b = pl.program_id(1)
    r0 = pl.multiple_of(b * RBK, RBK)

    @pl.when(l == 0)
    def _():
        s1[pl.ds(r0, RBK)] = x_ref[...]

    def layer(rb_ref, relu_out):
        h_full = rb_ref[...]
        a_bf = a_ref[...].astype(jnp.bfloat16)
        h_hi = h_full.astype(jnp.bfloat16)
        h_lo = (h_full - h_hi.astype(_f32)).astype(jnp.bfloat16)
        agg = (jnp.dot(a_bf, h_hi, preferred_element_type=_f32)
               + jnp.dot(a_bf, h_lo, preferred_element_type=_f32))
        z = rb_ref[pl.ds(r0, RBK)] + agg
        z1 = jnp.maximum(
            jnp.dot(z, w1_ref[0], preferred_element_type=_f32) + b1_ref[0],
            0.0)
        z2 = jnp.dot(z1, w2_ref[0], preferred_element_type=_f32) + b2_ref[0]
        return jnp.maximum(z2, 0.0) if relu_out else z2

    @pl.when(l == 1)
    def _():
        s0[pl.ds(r0, RBK)] = layer(s1, True)

    @pl.when(l == 2)
    def _():
        s1[pl.ds(r0, RBK)] = layer(s0, True)

    @pl.when(l == 3)
    def _():
        o_ref[...] = layer(s1, False)


def _tc_gnn(a, x, wg1, bg1, wg2, bg2):
    wmap = lambda l, b: (jnp.maximum(l - 1, 0), 0, 0)
    return pl.pallas_call(
        _tc_gnn_body,
        grid=(L + 1, NBLK),
        in_specs=[
            pl.BlockSpec((RBK, N), lambda l, b: (jnp.where(l == 0, 0, b), 0)),
            pl.BlockSpec((RBK, D), lambda l, b: (jnp.where(l == 0, b, 0), 0)),
            pl.BlockSpec((1, D, D), wmap),
            pl.BlockSpec((1, 1, D), wmap),
            pl.BlockSpec((1, D, D), wmap),
            pl.BlockSpec((1, 1, D), wmap),
        ],
        out_specs=pl.BlockSpec((RBK, D), lambda l, b: (b, 0)),
        out_shape=jax.ShapeDtypeStruct((N, D), _f32),
        scratch_shapes=[
            pltpu.VMEM((N, D), _f32),
            pltpu.VMEM((N, D), _f32),
        ],
    )(a, x, wg1, bg1, wg2, bg2)


def _tc_head_body(nf_ref, batch_ref, pos_ref, wm1_ref, bm1_ref, wm2_ref,
                  bm2_ref, pp_ref, gf_ref, ploss_ref, u_ref, w_ref):
    nf = nf_ref[...]
    t = jnp.dot(nf, wm1_ref[...], preferred_element_type=_f32) + bm1_ref[...]
    t = jnp.maximum(t, 0.0)
    pr = jnp.dot(t, wm2_ref[...], preferred_element_type=_f32) + bm2_ref[...]
    mu = jnp.mean(pr, axis=0, keepdims=True)
    y0 = pr - mu
    rms = jnp.sqrt(jnp.mean(y0 * y0))
    y = jnp.where(rms < 1e-8, y0, y0 * (1.0 / jnp.maximum(rms, 1e-8)))
    pp_ref[...] = y

    b_row = batch_ref[...]
    gids = lax.broadcasted_iota(jnp.int32, (G, N), 0)
    onehot = (gids == b_row).astype(_f32)
    cnt = jnp.sum(onehot, axis=1, keepdims=True)
    sums = jnp.dot(onehot, nf, preferred_element_type=_f32)
    gf_ref[...] = sums / jnp.maximum(cnt, 1.0)

    dpos = y - pos_ref[...]
    ploss_ref[0, 0] = jnp.sum(dpos * dpos) * (1.0 / (N * 3))

    sq = jnp.sum(y * y, axis=1, keepdims=True)
    ones = jnp.ones_like(sq)
    zer3 = jnp.zeros((N, 3), _f32)
    u_ref[...] = jnp.concatenate([-2.0 * y, ones, sq, zer3], axis=1)
    w_ref[...] = jnp.concatenate([y, sq, ones, zer3], axis=1)


def _tc_head(nf, batch_row, pos, wm1, bm1, wm2, bm2):
    return pl.pallas_call(
        _tc_head_body,
        out_shape=(
            jax.ShapeDtypeStruct((N, 3), _f32),
            jax.ShapeDtypeStruct((G, D), _f32),
            jax.ShapeDtypeStruct((1, 1), _f32),
            jax.ShapeDtypeStruct((N, 8), _f32),
            jax.ShapeDtypeStruct((N, 8), _f32),
        ),
        out_specs=(
            pl.BlockSpec((N, 3), lambda: (0, 0)),
            pl.BlockSpec((G, D), lambda: (0, 0)),
            pl.BlockSpec(memory_space=pltpu.SMEM),
            pl.BlockSpec((N, 8), lambda: (0, 0)),
            pl.BlockSpec((N, 8), lambda: (0, 0)),
        ),
    )(nf, batch_row, pos, wm1, bm1, wm2, bm2)


RB = 256
NBLK = N // RB


def _tc_loss_body(u_ref, w_ref, p_ref, o_ref):
    i = pl.program_id(0)
    d2 = lax.dot_general(u_ref[...], w_ref[...], (((1,), (1,)), ((), ())),
                         preferred_element_type=_f32)
    d2 = jnp.maximum(d2, 0.0)
    q = 1.0 / (1.0 + (d2 + EPS))
    cols = lax.broadcasted_iota(jnp.int32, (RB, N), 1)
    rows = lax.broadcasted_iota(jnp.int32, (RB, N), 0) + i * RB
    q = jnp.where(rows == cols, 0.0, q)
    p = p_ref[...]
    ce = -p * jnp.log(q + LOG_EPS) - (1.0 - p) * jnp.log(1.0 - q + LOG_EPS)
    part = jnp.sum(ce)

    @pl.when(i == 0)
    def _():
        o_ref[0, 0] = 0.0

    o_ref[0, 0] += part


def _tc_loss(u, w, p):
    return pl.pallas_call(
        _tc_loss_body,
        grid=(NBLK,),
        in_specs=[
            pl.BlockSpec((RB, 8), lambda i: (i, 0)),
            pl.BlockSpec((N, 8), lambda i: (0, 0)),
            pl.BlockSpec((RB, N), lambda i: (i, 0)),
        ],
        out_specs=pl.BlockSpec(memory_space=pltpu.SMEM),
        out_shape=jax.ShapeDtypeStruct((1, 1), _f32),
    )(u, w, p)


def kernel(x, pos, P, Wg1, bg1, Wg2, bg2, Wm1, bm1, Wm2, bm2,
           edge_index, batch, epoch):
    batch_row = batch.reshape(1, N)

    a = _sc_build_a(edge_index)
    h = _tc_gnn(a, x, Wg1, bg1.reshape(L, 1, D), Wg2, bg2.reshape(L, 1, D))

    pp, gf, ploss, u, w = _tc_head(h, batch_row, pos, Wm1,
                                   bm1.reshape(1, D), Wm2, bm2.reshape(1, 3))
    mani = _tc_loss(u, w, P)
    return (pp, gf, ploss.reshape(()), mani.reshape(()))


# speedup vs baseline: 1.0310x; 1.0310x over previous
import functools

import jax
import jax.numpy as jnp
from jax import lax
from jax.experimental import pallas as pl
from jax.experimental.pallas import tpu as pltpu
from jax.experimental.pallas import tpu_sc as plsc

N = 4096
E = 131072
D = 256
L = 3
G = 64
EPS = 1e-9
LOG_EPS = 1e-4

NC = 2
NS = 16
NW = NC * NS

RA = 24
PASS_STRIDE = RA * NW
NPASS = 6
ECHUNK = 4096
NECHUNK = E // ECHUNK

_f32 = jnp.float32


def _sc_build_a_body(edge_hbm, a_hbm, hist, ebuf0, ebuf1, esem0, esem1):
    c = lax.axis_index("c")
    s = lax.axis_index("s")
    gid = s * NC + c
    ones = jnp.full((16,), 1.0, _f32)
    z16 = jnp.zeros((16,), _f32)
    ebufs = (ebuf0, ebuf1)
    esems = (esem0, esem1)

    def chunk_off(ch):
        return pl.multiple_of(((ch + gid) % NECHUNK) * ECHUNK, 8)

    def start_chunk(ch, b):
        pltpu.make_async_copy(
            edge_hbm.at[:, pl.ds(chunk_off(ch), ECHUNK)], ebufs[b],
            esems[b]).start()

    def wait_chunk(ch, b):
        pltpu.make_async_copy(
            edge_hbm.at[:, pl.ds(chunk_off(ch), ECHUNK)], ebufs[b],
            esems[b]).wait()

    def one_pass(p, _):
        lo = jnp.minimum(p * PASS_STRIDE + gid * RA, N - RA)
        for r in range(RA):
            def zb(j, _2, r=r):
                for u in range(8):
                    hist[r, pl.ds(j * 128 + u * 16, 16)] = z16
                return 0
            lax.fori_loop(0, N // 128, zb, 0)
        start_chunk(0, 0)
        for ch in range(NECHUNK):
            b = ch % 2
            if ch + 1 < NECHUNK:
                start_chunk(ch + 1, (ch + 1) % 2)
            wait_chunk(ch, b)
            ebuf = ebufs[b]

            def scan(k, _2, ebuf=ebuf):
                for u in range(4):
                    s16 = ebuf[0, pl.ds(k * 64 + u * 16, 16)]
                    d16 = ebuf[1, pl.ds(k * 64 + u * 16, 16)]
                    rel = d16 - lo
                    mask = rel.astype(jnp.uint32) < jnp.uint32(RA)
                    plsc.addupdate_scatter(hist, [rel, s16], ones, mask=mask)
                return 0
            lax.fori_loop(0, ECHUNK // 64, scan, 0)
        pltpu.sync_copy(hist, a_hbm.at[pl.ds(lo, RA)])
        return 0

    lax.fori_loop(0, NPASS, one_pass, 0)


@functools.lru_cache(maxsize=1)
def _sc_build_a_kernel():
    return pl.kernel(
        _sc_build_a_body,
        out_type=jax.ShapeDtypeStruct((N, N), _f32),
        mesh=plsc.VectorSubcoreMesh(core_axis_name="c", subcore_axis_name="s",
                                    num_cores=NC, num_subcores=NS),
        compiler_params=pltpu.CompilerParams(needs_layout_passes=False),
        scratch_types=[
            pltpu.VMEM((RA, N), _f32),
            pltpu.VMEM((2, ECHUNK), jnp.int32),
            pltpu.VMEM((2, ECHUNK), jnp.int32),
            pltpu.SemaphoreType.DMA,
            pltpu.SemaphoreType.DMA,
        ],
    )


def _sc_build_a(edge_index):
    return _sc_build_a_kernel()(edge_index)


RBK = 512
NBLK = N // RBK


def _tc_gnn_body(a_ref, x_ref, w1_ref, b1_ref, w2_ref, b2_ref, o_ref, s0, s1):
    l = pl.program_id(0)
    b = pl.program_id(1)
    r0 = pl.multiple_of(b * RBK, RBK)

    @pl.when(l == 0)
    def _():
        s1[pl.ds(r0, RBK)] = x_ref[...]

    def layer(rb_ref, relu_out):
        h_full = rb_ref[...]
        agg = jnp.dot(a_ref[...], h_full, preferred_element_type=_f32)
        z = rb_ref[pl.ds(r0, RBK)] + agg
        z1 = jnp.maximum(
            jnp.dot(z, w1_ref[0], preferred_element_type=_f32) + b1_ref[0],
            0.0)
        z2 = jnp.dot(z1, w2_ref[0], preferred_element_type=_f32) + b2_ref[0]
        return jnp.maximum(z2, 0.0) if relu_out else z2

    @pl.when(l == 1)
    def _():
        s0[pl.ds(r0, RBK)] = layer(s1, True)

    @pl.when(l == 2)
    def _():
        s1[pl.ds(r0, RBK)] = layer(s0, True)

    @pl.when(l == 3)
    def _():
        o_ref[...] = layer(s1, False)


def _tc_gnn(a, x, wg1, bg1, wg2, bg2):
    wmap = lambda l, b: (jnp.maximum(l - 1, 0), 0, 0)
    return pl.pallas_call(
        _tc_gnn_body,
        grid=(L + 1, NBLK),
        in_specs=[
            pl.BlockSpec((RBK, N), lambda l, b: (jnp.where(l == 0, 0, b), 0)),
            pl.BlockSpec((RBK, D), lambda l, b: (jnp.where(l == 0, b, 0), 0)),
            pl.BlockSpec((1, D, D), wmap),
            pl.BlockSpec((1, 1, D), wmap),
            pl.BlockSpec((1, D, D), wmap),
            pl.BlockSpec((1, 1, D), wmap),
        ],
        out_specs=pl.BlockSpec((RBK, D), lambda l, b: (b, 0)),
        out_shape=jax.ShapeDtypeStruct((N, D), _f32),
        scratch_shapes=[
            pltpu.VMEM((N, D), _f32),
            pltpu.VMEM((N, D), _f32),
        ],
    )(a, x, wg1, bg1, wg2, bg2)


def _tc_head_body(nf_ref, batch_ref, pos_ref, wm1_ref, bm1_ref, wm2_ref,
                  bm2_ref, pp_ref, gf_ref, ploss_ref, u_ref, w_ref):
    nf = nf_ref[...]
    t = jnp.dot(nf, wm1_ref[...], preferred_element_type=_f32) + bm1_ref[...]
    t = jnp.maximum(t, 0.0)
    pr = jnp.dot(t, wm2_ref[...], preferred_element_type=_f32) + bm2_ref[...]
    mu = jnp.mean(pr, axis=0, keepdims=True)
    y0 = pr - mu
    rms = jnp.sqrt(jnp.mean(y0 * y0))
    y = jnp.where(rms < 1e-8, y0, y0 * (1.0 / jnp.maximum(rms, 1e-8)))
    pp_ref[...] = y

    b_row = batch_ref[...]
    gids = lax.broadcasted_iota(jnp.int32, (G, N), 0)
    onehot = (gids == b_row).astype(_f32)
    cnt = jnp.sum(onehot, axis=1, keepdims=True)
    sums = jnp.dot(onehot, nf, preferred_element_type=_f32)
    gf_ref[...] = sums / jnp.maximum(cnt, 1.0)

    dpos = y - pos_ref[...]
    ploss_ref[0, 0] = jnp.sum(dpos * dpos) * (1.0 / (N * 3))

    sq = jnp.sum(y * y, axis=1, keepdims=True)
    ones = jnp.ones_like(sq)
    zer3 = jnp.zeros((N, 3), _f32)
    u_ref[...] = jnp.concatenate([-2.0 * y, ones, sq, zer3], axis=1)
    w_ref[...] = jnp.concatenate([y, sq, ones, zer3], axis=1)


def _tc_head(nf, batch_row, pos, wm1, bm1, wm2, bm2):
    return pl.pallas_call(
        _tc_head_body,
        out_shape=(
            jax.ShapeDtypeStruct((N, 3), _f32),
            jax.ShapeDtypeStruct((G, D), _f32),
            jax.ShapeDtypeStruct((1, 1), _f32),
            jax.ShapeDtypeStruct((N, 8), _f32),
            jax.ShapeDtypeStruct((N, 8), _f32),
        ),
        out_specs=(
            pl.BlockSpec((N, 3), lambda: (0, 0)),
            pl.BlockSpec((G, D), lambda: (0, 0)),
            pl.BlockSpec(memory_space=pltpu.SMEM),
            pl.BlockSpec((N, 8), lambda: (0, 0)),
            pl.BlockSpec((N, 8), lambda: (0, 0)),
        ),
    )(nf, batch_row, pos, wm1, bm1, wm2, bm2)


RB = 256
NBLK = N // RB


def _tc_loss_body(u_ref, w_ref, p_ref, o_ref):
    i = pl.program_id(0)
    d2 = lax.dot_general(u_ref[...], w_ref[...], (((1,), (1,)), ((), ())),
                         preferred_element_type=_f32)
    d2 = jnp.maximum(d2, 0.0)
    q = 1.0 / (1.0 + (d2 + EPS))
    cols = lax.broadcasted_iota(jnp.int32, (RB, N), 1)
    rows = lax.broadcasted_iota(jnp.int32, (RB, N), 0) + i * RB
    q = jnp.where(rows == cols, 0.0, q)
    p = p_ref[...]
    ce = -p * jnp.log(q + LOG_EPS) - (1.0 - p) * jnp.log(1.0 - q + LOG_EPS)
    part = jnp.sum(ce)

    @pl.when(i == 0)
    def _():
        o_ref[0, 0] = 0.0

    o_ref[0, 0] += part


def _tc_loss(u, w, p):
    return pl.pallas_call(
        _tc_loss_body,
        grid=(NBLK,),
        in_specs=[
            pl.BlockSpec((RB, 8), lambda i: (i, 0)),
            pl.BlockSpec((N, 8), lambda i: (0, 0)),
            pl.BlockSpec((RB, N), lambda i: (i, 0)),
        ],
        out_specs=pl.BlockSpec(memory_space=pltpu.SMEM),
        out_shape=jax.ShapeDtypeStruct((1, 1), _f32),
    )(u, w, p)


def kernel(x, pos, P, Wg1, bg1, Wg2, bg2, Wm1, bm1, Wm2, bm2,
           edge_index, batch, epoch):
    batch_row = batch.reshape(1, N)

    a = _sc_build_a(edge_index)
    h = _tc_gnn(a, x, Wg1, bg1.reshape(L, 1, D), Wg2, bg2.reshape(L, 1, D))

    pp, gf, ploss, u, w = _tc_head(h, batch_row, pos, Wm1,
                                   bm1.reshape(1, D), Wm2, bm2.reshape(1, 3))
    mani = _tc_loss(u, w, P)
    return (pp, gf, ploss.reshape(()), mani.reshape(()))
